# Initial kernel scaffold; baseline (speedup 1.0000x reference)
import jax
import jax.numpy as jnp
from jax.experimental import pallas as pl


def kernel(x, a, e, W, b, attw, gnw, gnb, Wh, bh, hnw, hnb):
    raise NotImplementedError("write your pallas kernel here")


# speedup vs baseline: 15.3494x; 15.3494x over previous
import functools

import jax
import jax.numpy as jnp
from jax import lax
from jax.experimental import pallas as pl
from jax.experimental.pallas import tpu as pltpu
from jax.experimental.pallas import tpu_sc as plsc

_EPS = 1e-5
_NC = 2
_NS = 16
_NW = _NC * _NS
_LANES = 16


@functools.lru_cache(maxsize=None)
def _make_edge_kernel(n_nodes: int, n_edges: int):
    assert n_edges % (_NW * _LANES) == 0
    assert n_nodes % _LANES == 0
    epw = n_edges // _NW
    steps = epw // _LANES
    nzero = n_nodes // _LANES
    mesh = plsc.VectorSubcoreMesh(
        core_axis_name="c", subcore_axis_name="s",
        num_cores=_NC, num_subcores=_NS)

    @functools.partial(
        pl.kernel,
        out_type=jax.ShapeDtypeStruct((_NW, n_nodes), jnp.float32),
        mesh=mesh,
        compiler_params=pltpu.CompilerParams(needs_layout_passes=False),
        scratch_types=[
            pltpu.VMEM((2 * epw,), jnp.int32),
            pltpu.VMEM((n_nodes,), jnp.float32),
            pltpu.VMEM((n_nodes,), jnp.float32),
            pltpu.VMEM((n_nodes,), jnp.float32),
        ],
    )
    def edge_kernel(u_hbm, v_hbm, e_hbm, out_hbm, e_v, u_v, v_v, acc_v):
        wid = lax.axis_index("s") * _NC + lax.axis_index("c")
        base = wid * epw
        pltpu.sync_copy(e_hbm.at[pl.ds(2 * base, 2 * epw)], e_v)
        pltpu.sync_copy(u_hbm, u_v)
        pltpu.sync_copy(v_hbm, v_v)

        lanes = lax.iota(jnp.int32, _LANES)
        zeros_f = jnp.zeros((_LANES,), jnp.float32)

        def zero_body(i, carry):
            acc_v[pl.ds(i * _LANES, _LANES)] = zeros_f
            return carry

        lax.fori_loop(0, nzero, zero_body, 0)

        def body(i, carry):
            row = 2 * (i * _LANES + lanes)
            s_idx = plsc.load_gather(e_v, [row])
            d_idx = plsc.load_gather(e_v, [row + 1])
            uu = plsc.load_gather(u_v, [s_idx])
            vv = plsc.load_gather(v_v, [d_idx])
            logit = uu + vv
            logit = jnp.where(logit >= 0.0, logit, logit * 0.2)
            att = jnp.exp(logit)
            plsc.addupdate_scatter(acc_v, [d_idx], att)
            return carry

        lax.fori_loop(0, steps, body, 0)
        pltpu.sync_copy(acc_v, out_hbm.at[wid])

    return edge_kernel


def _uv_body(x_ref, w_ref, b_ref, attw_ref, u_ref, v_ref):
    d = x_ref.shape[1]
    a1 = attw_ref[:, :d]
    a2 = attw_ref[:, d:]
    w = w_ref[...]
    p = lax.dot_general(a1, w, (((1,), (0,)), ((), ())),
                        preferred_element_type=jnp.float32)
    q = lax.dot_general(a2, w, (((1,), (0,)), ((), ())),
                        preferred_element_type=jnp.float32)
    c = jnp.sum((a1 + a2) * b_ref[...])
    x = x_ref[...]
    u_ref[...] = lax.dot_general(p, x, (((1,), (1,)), ((), ())),
                                 preferred_element_type=jnp.float32) + c
    v_ref[...] = lax.dot_general(q, x, (((1,), (1,)), ((), ())),
                                 preferred_element_type=jnp.float32)


def _rms(t, w, bias):
    inv = lax.rsqrt(jnp.mean(t * t, axis=-1, keepdims=True) + _EPS)
    return t * inv * w + bias


def _dense_body(x_ref, part_ref, gnw_ref, gnb_ref, wh_ref, bh_ref,
                hnw_ref, hnb_ref, wn_ref, bn_ref, attwn_ref,
                x_out_ref, u_ref, v_ref):
    d = x_ref.shape[1]
    x = x_ref[...]
    nsum = jnp.sum(part_ref[...], axis=0)
    nsum = jnp.where(nsum == 0.0, 1.0, nsum)
    mm = jnp.maximum(x / nsum[:, None], 0.0)
    x1 = _rms(mm + x, gnw_ref[...], gnb_ref[...])
    y = lax.dot_general(x1, wh_ref[...], (((1,), (1,)), ((), ())),
                        preferred_element_type=jnp.float32) + bh_ref[...]
    y = jnp.maximum(y, 0.0)
    x2 = _rms(x1 + y, hnw_ref[...], hnb_ref[...])
    x_out_ref[...] = x2
    a1 = attwn_ref[:, :d]
    a2 = attwn_ref[:, d:]
    wn = wn_ref[...]
    p = lax.dot_general(a1, wn, (((1,), (0,)), ((), ())),
                        preferred_element_type=jnp.float32)
    q = lax.dot_general(a2, wn, (((1,), (0,)), ((), ())),
                        preferred_element_type=jnp.float32)
    c = jnp.sum((a1 + a2) * bn_ref[...])
    u_ref[...] = lax.dot_general(p, x2, (((1,), (1,)), ((), ())),
                                 preferred_element_type=jnp.float32) + c
    v_ref[...] = lax.dot_general(q, x2, (((1,), (1,)), ((), ())),
                                 preferred_element_type=jnp.float32)


def kernel(x, a, e, W, b, attw, gnw, gnb, Wh, bh, hnw, hnb):
    del a
    n, d = x.shape
    n_edges = e.shape[0]
    n_layers = W.shape[0]

    edge_call = _make_edge_kernel(n, n_edges)

    uv_call = pl.pallas_call(
        _uv_body,
        out_shape=(jax.ShapeDtypeStruct((1, n), jnp.float32),
                   jax.ShapeDtypeStruct((1, n), jnp.float32)),
    )
    dense_call = pl.pallas_call(
        _dense_body,
        out_shape=(jax.ShapeDtypeStruct((n, d), jnp.float32),
                   jax.ShapeDtypeStruct((1, n), jnp.float32),
                   jax.ShapeDtypeStruct((1, n), jnp.float32)),
    )

    e_flat = e.reshape(-1)
    u, v = uv_call(x, W[0], b[0][None, :], attw[0])
    for i in range(n_layers):
        part = edge_call(u.reshape(n), v.reshape(n), e_flat)
        j = min(i + 1, n_layers - 1)
        x, u, v = dense_call(x, part, gnw[i][None, :], gnb[i][None, :],
                             Wh[i], bh[i][None, :], hnw[i][None, :],
                             hnb[i][None, :], W[j], b[j][None, :], attw[j])
    return x


# speedup vs baseline: 17.4989x; 1.1400x over previous
import functools

import jax
import jax.numpy as jnp
from jax import lax
from jax.experimental import pallas as pl
from jax.experimental.pallas import tpu as pltpu
from jax.experimental.pallas import tpu_sc as plsc

_EPS = 1e-5
_NC = 2
_NS = 16
_NW = _NC * _NS
_LANES = 16


@functools.lru_cache(maxsize=None)
def _make_edge_kernel(n_nodes: int, n_edges: int):
    assert n_edges % (_NW * _LANES) == 0
    assert n_nodes % _LANES == 0
    epw = n_edges // _NW
    steps = epw // _LANES
    nzero = n_nodes // _LANES
    mesh = plsc.VectorSubcoreMesh(
        core_axis_name="c", subcore_axis_name="s",
        num_cores=_NC, num_subcores=_NS)

    @functools.partial(
        pl.kernel,
        out_type=jax.ShapeDtypeStruct((_NW, n_nodes), jnp.float32),
        mesh=mesh,
        compiler_params=pltpu.CompilerParams(needs_layout_passes=False),
        scratch_types=[
            pltpu.VMEM((2 * epw,), jnp.int32),
            pltpu.VMEM((n_nodes,), jnp.float32),
            pltpu.VMEM((n_nodes,), jnp.float32),
            pltpu.VMEM((n_nodes,), jnp.float32),
        ],
    )
    def edge_kernel(u_hbm, v_hbm, e_hbm, out_hbm, e_v, u_v, v_v, acc_v):
        wid = lax.axis_index("s") * _NC + lax.axis_index("c")
        base = wid * epw
        pltpu.sync_copy(e_hbm.at[pl.ds(2 * base, 2 * epw)], e_v)
        pltpu.sync_copy(u_hbm.at[0], u_v)
        pltpu.sync_copy(v_hbm.at[0], v_v)

        lanes = lax.iota(jnp.int32, _LANES)
        zeros_f = jnp.zeros((_LANES,), jnp.float32)

        @plsc.parallel_loop(0, nzero, unroll=8)
        def zero_body(i):
            acc_v[pl.ds(i * _LANES, _LANES)] = zeros_f

        @plsc.parallel_loop(0, steps, unroll=8)
        def body(i):
            row = 2 * (i * _LANES + lanes)
            s_idx = plsc.load_gather(e_v, [row])
            d_idx = plsc.load_gather(e_v, [row + 1])
            uu = plsc.load_gather(u_v, [s_idx])
            vv = plsc.load_gather(v_v, [d_idx])
            logit = uu + vv
            logit = jnp.where(logit >= 0.0, logit, logit * 0.2)
            att = jnp.exp(logit)
            plsc.addupdate_scatter(acc_v, [d_idx], att)

        pltpu.sync_copy(acc_v, out_hbm.at[wid])

    return edge_kernel


def _uv_body(x_ref, w_ref, b_ref, attw_ref, u_ref, v_ref, *, layer):
    d = x_ref.shape[1]
    a1 = attw_ref[layer, :, :d]
    a2 = attw_ref[layer, :, d:]
    w = w_ref[layer]
    b_row = b_ref[layer][None, :]
    p = lax.dot_general(a1, w, (((1,), (0,)), ((), ())),
                        preferred_element_type=jnp.float32)
    q = lax.dot_general(a2, w, (((1,), (0,)), ((), ())),
                        preferred_element_type=jnp.float32)
    c = jnp.sum((a1 + a2) * b_row)
    x = x_ref[...]
    u_ref[...] = lax.dot_general(p, x, (((1,), (1,)), ((), ())),
                                 preferred_element_type=jnp.float32) + c
    v_ref[...] = lax.dot_general(q, x, (((1,), (1,)), ((), ())),
                                 preferred_element_type=jnp.float32)


def _rms(t, w, bias):
    inv = lax.rsqrt(jnp.mean(t * t, axis=-1, keepdims=True) + _EPS)
    return t * inv * w + bias


def _dense_body(x_ref, part_ref, gnw_ref, gnb_ref, wh_ref, bh_ref,
                hnw_ref, hnb_ref, wn_ref, bn_ref, attwn_ref,
                x_out_ref, u_ref, v_ref, *, layer, nxt):
    d = x_ref.shape[1]
    x = x_ref[...]
    nsum = jnp.sum(part_ref[...], axis=0)
    nsum = jnp.where(nsum == 0.0, 1.0, nsum)
    mm = jnp.maximum(x / nsum[:, None], 0.0)
    x1 = _rms(mm + x, gnw_ref[layer][None, :], gnb_ref[layer][None, :])
    y = lax.dot_general(x1, wh_ref[layer], (((1,), (1,)), ((), ())),
                        preferred_element_type=jnp.float32) + bh_ref[layer][None, :]
    y = jnp.maximum(y, 0.0)
    x2 = _rms(x1 + y, hnw_ref[layer][None, :], hnb_ref[layer][None, :])
    x_out_ref[...] = x2
    a1 = attwn_ref[nxt, :, :d]
    a2 = attwn_ref[nxt, :, d:]
    wn = wn_ref[nxt]
    p = lax.dot_general(a1, wn, (((1,), (0,)), ((), ())),
                        preferred_element_type=jnp.float32)
    q = lax.dot_general(a2, wn, (((1,), (0,)), ((), ())),
                        preferred_element_type=jnp.float32)
    c = jnp.sum((a1 + a2) * bn_ref[nxt][None, :])
    u_ref[...] = lax.dot_general(p, x2, (((1,), (1,)), ((), ())),
                                 preferred_element_type=jnp.float32) + c
    v_ref[...] = lax.dot_general(q, x2, (((1,), (1,)), ((), ())),
                                 preferred_element_type=jnp.float32)


def kernel(x, a, e, W, b, attw, gnw, gnb, Wh, bh, hnw, hnb):
    del a
    n, d = x.shape
    n_edges = e.shape[0]
    n_layers = W.shape[0]

    edge_call = _make_edge_kernel(n, n_edges)

    uv_out = (jax.ShapeDtypeStruct((1, n), jnp.float32),
              jax.ShapeDtypeStruct((1, n), jnp.float32))
    uv_call = pl.pallas_call(functools.partial(_uv_body, layer=0),
                             out_shape=uv_out)

    e_flat = e.reshape(-1)
    u, v = uv_call(x, W, b, attw)
    for i in range(n_layers):
        part = edge_call(u, v, e_flat)
        j = min(i + 1, n_layers - 1)
        dense_call = pl.pallas_call(
            functools.partial(_dense_body, layer=i, nxt=j),
            out_shape=(jax.ShapeDtypeStruct((n, d), jnp.float32),) + uv_out,
        )
        x, u, v = dense_call(x, part, gnw, gnb, Wh, bh, hnw, hnb, W, b, attw)
    return x


# speedup vs baseline: 55.1822x; 3.1535x over previous
import functools

import jax
import jax.numpy as jnp
from jax import lax
from jax.experimental import pallas as pl
from jax.experimental.pallas import tpu as pltpu
from jax.experimental.pallas import tpu_sc as plsc

_EPS = 1e-5
_NC = 2
_NS = 16
_NW = _NC * _NS
_LANES = 16


@functools.lru_cache(maxsize=None)
def _make_edge_kernel(n_nodes: int, n_edges: int):
    assert n_edges % (_NW * _LANES) == 0
    assert n_nodes % _LANES == 0
    epw = n_edges // _NW
    steps = epw // _LANES
    nzero = n_nodes // _LANES
    mesh = plsc.VectorSubcoreMesh(
        core_axis_name="c", subcore_axis_name="s",
        num_cores=_NC, num_subcores=_NS)

    @functools.partial(
        pl.kernel,
        out_type=jax.ShapeDtypeStruct((_NW, n_nodes), jnp.float32),
        mesh=mesh,
        compiler_params=pltpu.CompilerParams(needs_layout_passes=False),
        scratch_types=[
            pltpu.VMEM((epw,), jnp.int32),
            pltpu.VMEM((epw,), jnp.int32),
            pltpu.VMEM((n_nodes,), jnp.float32),
            pltpu.VMEM((n_nodes,), jnp.float32),
            pltpu.VMEM((n_nodes,), jnp.float32),
        ],
    )
    def edge_kernel(u_hbm, v_hbm, s_hbm, d_hbm, out_hbm, s_v, d_v, u_v, v_v,
                    acc_v):
        wid = lax.axis_index("s") * _NC + lax.axis_index("c")
        base = wid * epw
        pltpu.sync_copy(s_hbm.at[pl.ds(base, epw)], s_v)
        pltpu.sync_copy(d_hbm.at[pl.ds(base, epw)], d_v)
        pltpu.sync_copy(u_hbm.at[0], u_v)
        pltpu.sync_copy(v_hbm.at[0], v_v)

        zeros_f = jnp.zeros((_LANES,), jnp.float32)

        @plsc.parallel_loop(0, nzero, unroll=8)
        def zero_body(i):
            acc_v[pl.ds(i * _LANES, _LANES)] = zeros_f

        @plsc.parallel_loop(0, steps, unroll=8)
        def body(i):
            s_idx = s_v[pl.ds(i * _LANES, _LANES)]
            d_idx = d_v[pl.ds(i * _LANES, _LANES)]
            uu = plsc.load_gather(u_v, [s_idx])
            vv = plsc.load_gather(v_v, [d_idx])
            logit = uu + vv
            logit = jnp.where(logit >= 0.0, logit, logit * 0.2)
            att = jnp.exp(logit)
            plsc.addupdate_scatter(acc_v, [d_idx], att)

        pltpu.sync_copy(acc_v, out_hbm.at[wid])

    return edge_kernel


def _uv_body(x_ref, w_ref, b_ref, attw_ref, u_ref, v_ref, *, layer):
    d = x_ref.shape[1]
    a1 = attw_ref[layer, :, :d]
    a2 = attw_ref[layer, :, d:]
    w = w_ref[layer]
    b_row = b_ref[layer][None, :]
    p = lax.dot_general(a1, w, (((1,), (0,)), ((), ())),
                        preferred_element_type=jnp.float32)
    q = lax.dot_general(a2, w, (((1,), (0,)), ((), ())),
                        preferred_element_type=jnp.float32)
    c = jnp.sum((a1 + a2) * b_row)
    x = x_ref[...]
    u_ref[...] = lax.dot_general(p, x, (((1,), (1,)), ((), ())),
                                 preferred_element_type=jnp.float32) + c
    v_ref[...] = lax.dot_general(q, x, (((1,), (1,)), ((), ())),
                                 preferred_element_type=jnp.float32)


def _rms(t, w, bias):
    inv = lax.rsqrt(jnp.mean(t * t, axis=-1, keepdims=True) + _EPS)
    return t * inv * w + bias


def _dense_body(x_ref, part_ref, gnw_ref, gnb_ref, wh_ref, bh_ref,
                hnw_ref, hnb_ref, wn_ref, bn_ref, attwn_ref,
                x_out_ref, u_ref, v_ref, *, layer, nxt):
    d = x_ref.shape[1]
    x = x_ref[...]
    nsum = jnp.sum(part_ref[...], axis=0)
    nsum = jnp.where(nsum == 0.0, 1.0, nsum)
    mm = jnp.maximum(x / nsum[:, None], 0.0)
    x1 = _rms(mm + x, gnw_ref[layer][None, :], gnb_ref[layer][None, :])
    y = lax.dot_general(x1, wh_ref[layer], (((1,), (1,)), ((), ())),
                        preferred_element_type=jnp.float32) + bh_ref[layer][None, :]
    y = jnp.maximum(y, 0.0)
    x2 = _rms(x1 + y, hnw_ref[layer][None, :], hnb_ref[layer][None, :])
    x_out_ref[...] = x2
    a1 = attwn_ref[nxt, :, :d]
    a2 = attwn_ref[nxt, :, d:]
    wn = wn_ref[nxt]
    p = lax.dot_general(a1, wn, (((1,), (0,)), ((), ())),
                        preferred_element_type=jnp.float32)
    q = lax.dot_general(a2, wn, (((1,), (0,)), ((), ())),
                        preferred_element_type=jnp.float32)
    c = jnp.sum((a1 + a2) * bn_ref[nxt][None, :])
    u_ref[...] = lax.dot_general(p, x2, (((1,), (1,)), ((), ())),
                                 preferred_element_type=jnp.float32) + c
    v_ref[...] = lax.dot_general(q, x2, (((1,), (1,)), ((), ())),
                                 preferred_element_type=jnp.float32)


def kernel(x, a, e, W, b, attw, gnw, gnb, Wh, bh, hnw, hnb):
    del a
    n, d = x.shape
    n_edges = e.shape[0]
    n_layers = W.shape[0]

    edge_call = _make_edge_kernel(n, n_edges)

    uv_out = (jax.ShapeDtypeStruct((1, n), jnp.float32),
              jax.ShapeDtypeStruct((1, n), jnp.float32))
    uv_call = pl.pallas_call(functools.partial(_uv_body, layer=0),
                             out_shape=uv_out)

    e_src = e[:, 0]
    e_dst = e[:, 1]
    u, v = uv_call(x, W, b, attw)
    for i in range(n_layers):
        part = edge_call(u, v, e_src, e_dst)
        j = min(i + 1, n_layers - 1)
        dense_call = pl.pallas_call(
            functools.partial(_dense_body, layer=i, nxt=j),
            out_shape=(jax.ShapeDtypeStruct((n, d), jnp.float32),) + uv_out,
        )
        x, u, v = dense_call(x, part, gnw, gnb, Wh, bh, hnw, hnb, W, b, attw)
    return x
